# baseline (speedup 1.0000x reference)
import jax
import jax.numpy as jnp
import numpy as np
from jax import lax
from jax.experimental import pallas as pl
from jax.experimental.pallas import tpu as pltpu
from jax.experimental.pallas import tpu_sc as plsc

B, V = 64, 100000
TEMPERATURE = np.float32(0.8)
TOP_P = np.float32(0.95)
K = 50

BLK = 160
NBLK = V // BLK
NBLK_PAD = 640
VPAD = NBLK_PAD * BLK
NVEC_BLK = BLK // 16
NSUP = NBLK_PAD // 16
NTILES = 32
ROWS_PER_TILE = B // NTILES

NEG_INF = np.float32(-np.inf)
TINY = np.float32(np.finfo(np.float32).tiny)
BIG_I = np.int32(1 << 30)

LG1 = np.float32(0.66666662693)
LG2 = np.float32(0.40000972152)
LG3 = np.float32(0.28498786688)
LG4 = np.float32(0.24279078841)
LN2_HI = np.float32(6.9313812256e-01)
LN2_LO = np.float32(9.0580006145e-06)
SQRT2 = np.float32(1.4142135623730951)
FLT_MIN = np.float32(1.1754943508222875e-38)
TWO23 = np.float32(8388608.0)


def _f32(v):
  return jnp.full((16,), v, dtype=jnp.float32)


def _i32(v):
  return jnp.full((16,), v, dtype=jnp.int32)


def _u32(v):
  return jnp.full((16,), np.uint32(v), dtype=jnp.uint32)


def _log_f32(v):
  isden = v < _f32(FLT_MIN)
  vs = jnp.where(isden, v * _f32(TWO23), v)
  bits = lax.bitcast_convert_type(vs, jnp.uint32)
  e_raw = lax.convert_element_type(
      lax.shift_right_logical(bits, _u32(23)), jnp.int32) - _i32(127)
  mbits = lax.bitwise_or(
      lax.bitwise_and(bits, _u32(0x007FFFFF)), _u32(0x3F800000))
  m = lax.bitcast_convert_type(mbits, jnp.float32)
  big = m > _f32(SQRT2)
  m = jnp.where(big, m * _f32(0.5), m)
  e = e_raw + jnp.where(big, _i32(1), _i32(0))
  e = e - jnp.where(isden, _i32(23), _i32(0))
  ef = lax.convert_element_type(e, jnp.float32)
  f = m - _f32(1.0)
  s = f / (_f32(2.0) + f)
  z = s * s
  w = z * z
  t1 = w * (_f32(LG2) + w * _f32(LG4))
  t2 = z * (_f32(LG1) + w * _f32(LG3))
  r = t2 + t1
  hfsq = _f32(0.5) * f * f
  return ef * _f32(LN2_HI) - (
      (hfsq - (s * (hfsq + r) + ef * _f32(LN2_LO))) - f)


def _rotl(x, r):
  return lax.bitwise_or(
      lax.shift_left(x, _u32(r)), lax.shift_right_logical(x, _u32(32 - r)))


def _threefry_bits(n_u32):
  ks0 = _u32(0)
  ks1 = _u32(42)
  ks2 = _u32(np.uint32(0) ^ np.uint32(42) ^ np.uint32(0x1BD11BDA))
  r0 = (13, 15, 26, 6)
  r1 = (17, 29, 16, 24)
  x0 = ks0
  x1 = n_u32 + ks1

  def rounds(x0, x1, rs):
    for r in rs:
      x0 = x0 + x1
      x1 = _rotl(x1, r)
      x1 = lax.bitwise_xor(x1, x0)
    return x0, x1

  x0, x1 = rounds(x0, x1, r0)
  x0 = x0 + ks1
  x1 = x1 + ks2 + _u32(1)
  x0, x1 = rounds(x0, x1, r1)
  x0 = x0 + ks2
  x1 = x1 + ks0 + _u32(2)
  x0, x1 = rounds(x0, x1, r0)
  x0 = x0 + ks0
  x1 = x1 + ks1 + _u32(3)
  x0, x1 = rounds(x0, x1, r1)
  x0 = x0 + ks1
  x1 = x1 + ks2 + _u32(4)
  x0, x1 = rounds(x0, x1, r0)
  x0 = x0 + ks2
  x1 = x1 + ks0 + _u32(5)
  return lax.bitwise_xor(x0, x1)


def _gumbel(n_i32):
  bits = _threefry_bits(lax.bitcast_convert_type(n_i32, jnp.uint32))
  fb = lax.bitwise_or(lax.shift_right_logical(bits, _u32(9)), _u32(0x3F800000))
  f = lax.bitcast_convert_type(fb, jnp.float32) - _f32(1.0)
  u = jnp.maximum(_f32(TINY), f * _f32(1.0) + _f32(TINY))
  neg_log_u = -_log_f32(u)
  return -_log_f32(neg_log_u)


def _body(logits_hbm, out_hbm, row_v, blkmax_v, outv_v):
  cid = lax.axis_index("c")
  sid = lax.axis_index("s")
  wid = sid * 2 + cid
  iota = lax.iota(jnp.int32, 16)

  def blend_store(ref, idx, val):
    base = lax.shift_left(lax.shift_right_logical(idx, 4), 4)
    lane = idx - base
    vec = ref[pl.ds(base, 16)]
    ref[pl.ds(base, 16)] = jnp.where(iota == lane, val, vec)

  outv_v[pl.ds(0, 16)] = _i32(0)

  for j in range(ROWS_PER_TILE):
    r = wid * ROWS_PER_TILE + j
    pltpu.sync_copy(logits_hbm.at[r], row_v)

    @plsc.parallel_loop(0, (NBLK - 1) // 16, unroll=2)
    def _bm16_body(g):
      base0 = g * (16 * BLK)
      outv = _f32(NEG_INF)
      for u in range(16):
        acc = row_v[pl.ds(base0 + u * BLK, 16)]
        for t in range(1, NVEC_BLK):
          acc = jnp.maximum(acc, row_v[pl.ds(base0 + u * BLK + 16 * t, 16)])
        outv = jnp.where(iota == _i32(u), jnp.max(acc), outv)
      blkmax_v[pl.ds(g * 16, 16)] = outv / _f32(TEMPERATURE)

    acc = row_v[pl.ds((NBLK - 1) * BLK, 16)]
    for t in range(1, NVEC_BLK):
      acc = jnp.maximum(acc, row_v[pl.ds((NBLK - 1) * BLK + 16 * t, 16)])
    outv = jnp.where(iota == _i32(0), jnp.max(acc), _f32(NEG_INF))
    blkmax_v[pl.ds(NBLK - 1, 16)] = outv / _f32(TEMPERATURE)

    svs = []
    for h in range(3):
      outv = _f32(NEG_INF)
      for u in range(16):
        s = h * 16 + u
        if s < NSUP:
          outv = jnp.where(iota == _i32(u), jnp.max(blkmax_v[pl.ds(s * 16, 16)]),
                           outv)
      svs.append(outv)

    def tour_body(i, carry):
      sv0, sv1, sv2, cv0, cv1, cv2, cv3, ci0, ci1, ci2, ci3 = carry
      m = jnp.max(jnp.maximum(jnp.maximum(sv0, sv1), sv2))
      f0 = plsc.all_reduce_ffs(sv0 == m)
      f1 = plsc.all_reduce_ffs(sv1 == m)
      f2 = plsc.all_reduce_ffs(sv2 == m)
      a0 = jnp.where(f0 == _i32(16), _i32(BIG_I), f0)
      a1 = jnp.where(f1 == _i32(16), _i32(BIG_I), f1 + _i32(16))
      a2 = jnp.where(f2 == _i32(16), _i32(BIG_I), f2 + _i32(32))
      s_star = jnp.minimum(jnp.minimum(a0, a1), a2)

      bmv = plsc.load_gather(blkmax_v, [s_star * _i32(16) + iota])
      b_star = s_star * _i32(16) + plsc.all_reduce_ffs(bmv == m)

      base = b_star * _i32(BLK)
      xvs = []
      off = _i32(BIG_I)
      for t in range(NVEC_BLK):
        xv = plsc.load_gather(row_v, [base + _i32(16 * t) + iota])
        xv = xv / _f32(TEMPERATURE)
        xvs.append(xv)
        ft = plsc.all_reduce_ffs(xv == m)
        off = jnp.minimum(
            off, jnp.where(ft == _i32(16), _i32(BIG_I), ft + _i32(16 * t)))
      v_star = base + off

      plsc.store_scatter(row_v, [v_star], _f32(NEG_INF), mask=iota == _i32(0))
      accv = _f32(NEG_INF)
      for t in range(NVEC_BLK):
        accv = jnp.maximum(
            accv,
            jnp.where(iota + _i32(16 * t) == off, _f32(NEG_INF), xvs[t]))
      newmax = jnp.max(accv)
      suprest = jnp.max(
          jnp.where(iota + _i32(16) * s_star == b_star, _f32(NEG_INF), bmv))
      supnew = _f32(0.0) + jnp.maximum(suprest, newmax)
      plsc.store_scatter(blkmax_v, [b_star], _f32(0.0) + newmax,
                         mask=iota == _i32(0))
      sv0 = jnp.where(iota == s_star, supnew, sv0)
      sv1 = jnp.where(iota + _i32(16) == s_star, supnew, sv1)
      sv2 = jnp.where(iota + _i32(32) == s_star, supnew, sv2)

      mv = _f32(0.0) + m
      cv0 = jnp.where(iota == i, mv, cv0)
      cv1 = jnp.where(iota + _i32(16) == i, mv, cv1)
      cv2 = jnp.where(iota + _i32(32) == i, mv, cv2)
      cv3 = jnp.where(iota + _i32(48) == i, mv, cv3)
      ci0 = jnp.where(iota == i, v_star, ci0)
      ci1 = jnp.where(iota + _i32(16) == i, v_star, ci1)
      ci2 = jnp.where(iota + _i32(32) == i, v_star, ci2)
      ci3 = jnp.where(iota + _i32(48) == i, v_star, ci3)
      return (sv0, sv1, sv2, cv0, cv1, cv2, cv3, ci0, ci1, ci2, ci3)

    init = (svs[0], svs[1], svs[2],
            _f32(NEG_INF), _f32(NEG_INF), _f32(NEG_INF), _f32(NEG_INF),
            _i32(0), _i32(0), _i32(0), _i32(0))
    (_, _, _, cv0, cv1, cv2, cv3,
     ci0, ci1, ci2, ci3) = lax.fori_loop(0, K, tour_body, init)

    m0 = jnp.max(cv0)
    xs = [cv0, cv1, cv2, cv3]
    idxs = [ci0, ci1, ci2, ci3]
    lanes = [iota + _i32(16 * t) for t in range(4)]
    valid = [lanes[t] < _i32(K) for t in range(4)]

    es = []
    for t in range(4):
      x_safe = jnp.where(valid[t], xs[t], _f32(0.0) + m0)
      es.append(jnp.where(valid[t], jnp.exp(x_safe - m0), _f32(0.0)))
    s1 = jnp.sum(es[0]) + jnp.sum(es[1]) + jnp.sum(es[2]) + jnp.sum(es[3])

    kcount = jnp.int32(0)
    carry = jnp.float32(0.0)
    for t in range(4):
      p = es[t] / s1
      c = plsc.cumsum(p) + carry
      carry = jnp.max(c)
      cnt = plsc.all_reduce_population_count(c <= _f32(TOP_P))
      kcount = kcount + jnp.max(cnt)

    keeps = [jnp.logical_and(valid[t], lanes[t] <= kcount) for t in range(4)]
    e2s = [jnp.where(keeps[t], es[t], _f32(0.0)) for t in range(4)]
    s2 = jnp.sum(e2s[0]) + jnp.sum(e2s[1]) + jnp.sum(e2s[2]) + jnp.sum(e2s[3])

    best = _f32(NEG_INF)
    avs = []
    for t in range(4):
      q = e2s[t] / s2
      pos = q > _f32(0.0)
      q_safe = jnp.where(pos, q, _f32(1.0))
      logq = _log_f32(q_safe)
      g = _gumbel(idxs[t] + _i32(r * V))
      a = jnp.where(
          jnp.logical_and(keeps[t], pos), logq + g, _f32(NEG_INF))
      avs.append(a)
      best = jnp.maximum(best, a)
    bestv = jnp.max(best)
    win = _i32(BIG_I)
    for t in range(4):
      win = jnp.minimum(win, jnp.where(avs[t] == bestv, idxs[t], _i32(BIG_I)))
    blend_store(outv_v, jnp.int32(j), jnp.min(win))

  pltpu.sync_copy(outv_v, out_hbm.at[wid])


def kernel(logits_BCxV):
  mesh = plsc.VectorSubcoreMesh(core_axis_name="c", subcore_axis_name="s")
  run = pl.kernel(
      _body,
      out_type=jax.ShapeDtypeStruct((NTILES, 16), jnp.int32),
      mesh=mesh,
      compiler_params=pltpu.CompilerParams(needs_layout_passes=False),
      scratch_types=[
          pltpu.VMEM((V,), jnp.float32),
          pltpu.VMEM((NBLK_PAD,), jnp.float32),
          pltpu.VMEM((16,), jnp.int32),
      ],
  )
  res = run(logits_BCxV)
  return res[:, :ROWS_PER_TILE].reshape(B)


# speedup vs baseline: 1.0888x; 1.0888x over previous
import jax
import jax.numpy as jnp
import numpy as np
from jax import lax
from jax.experimental import pallas as pl
from jax.experimental.pallas import tpu as pltpu
from jax.experimental.pallas import tpu_sc as plsc

B, V = 64, 100000
TEMPERATURE = np.float32(0.8)
TOP_P = np.float32(0.95)
K = 50

BLK = 160
NBLK = V // BLK
NBLK_PAD = 640
VPAD = NBLK_PAD * BLK
NVEC_BLK = BLK // 16
NSUP = NBLK_PAD // 16
NTILES = 32
ROWS_PER_TILE = B // NTILES

NEG_INF = np.float32(-np.inf)
TINY = np.float32(np.finfo(np.float32).tiny)
BIG_I = np.int32(1 << 30)

LG1 = np.float32(0.66666662693)
LG2 = np.float32(0.40000972152)
LG3 = np.float32(0.28498786688)
LG4 = np.float32(0.24279078841)
LN2_HI = np.float32(6.9313812256e-01)
LN2_LO = np.float32(9.0580006145e-06)
SQRT2 = np.float32(1.4142135623730951)
FLT_MIN = np.float32(1.1754943508222875e-38)
TWO23 = np.float32(8388608.0)


def _f32(v):
  return jnp.full((16,), v, dtype=jnp.float32)


def _i32(v):
  return jnp.full((16,), v, dtype=jnp.int32)


def _u32(v):
  return jnp.full((16,), np.uint32(v), dtype=jnp.uint32)


def _log_f32(v):
  isden = v < _f32(FLT_MIN)
  vs = jnp.where(isden, v * _f32(TWO23), v)
  bits = lax.bitcast_convert_type(vs, jnp.uint32)
  e_raw = lax.convert_element_type(
      lax.shift_right_logical(bits, _u32(23)), jnp.int32) - _i32(127)
  mbits = lax.bitwise_or(
      lax.bitwise_and(bits, _u32(0x007FFFFF)), _u32(0x3F800000))
  m = lax.bitcast_convert_type(mbits, jnp.float32)
  big = m > _f32(SQRT2)
  m = jnp.where(big, m * _f32(0.5), m)
  e = e_raw + jnp.where(big, _i32(1), _i32(0))
  e = e - jnp.where(isden, _i32(23), _i32(0))
  ef = lax.convert_element_type(e, jnp.float32)
  f = m - _f32(1.0)
  s = f / (_f32(2.0) + f)
  z = s * s
  w = z * z
  t1 = w * (_f32(LG2) + w * _f32(LG4))
  t2 = z * (_f32(LG1) + w * _f32(LG3))
  r = t2 + t1
  hfsq = _f32(0.5) * f * f
  return ef * _f32(LN2_HI) - (
      (hfsq - (s * (hfsq + r) + ef * _f32(LN2_LO))) - f)


def _rotl(x, r):
  return lax.bitwise_or(
      lax.shift_left(x, _u32(r)), lax.shift_right_logical(x, _u32(32 - r)))


def _threefry_bits(n_u32):
  ks0 = _u32(0)
  ks1 = _u32(42)
  ks2 = _u32(np.uint32(0) ^ np.uint32(42) ^ np.uint32(0x1BD11BDA))
  r0 = (13, 15, 26, 6)
  r1 = (17, 29, 16, 24)
  x0 = ks0
  x1 = n_u32 + ks1

  def rounds(x0, x1, rs):
    for r in rs:
      x0 = x0 + x1
      x1 = _rotl(x1, r)
      x1 = lax.bitwise_xor(x1, x0)
    return x0, x1

  x0, x1 = rounds(x0, x1, r0)
  x0 = x0 + ks1
  x1 = x1 + ks2 + _u32(1)
  x0, x1 = rounds(x0, x1, r1)
  x0 = x0 + ks2
  x1 = x1 + ks0 + _u32(2)
  x0, x1 = rounds(x0, x1, r0)
  x0 = x0 + ks0
  x1 = x1 + ks1 + _u32(3)
  x0, x1 = rounds(x0, x1, r1)
  x0 = x0 + ks1
  x1 = x1 + ks2 + _u32(4)
  x0, x1 = rounds(x0, x1, r0)
  x0 = x0 + ks2
  x1 = x1 + ks0 + _u32(5)
  return lax.bitwise_xor(x0, x1)


def _gumbel(n_i32):
  bits = _threefry_bits(lax.bitcast_convert_type(n_i32, jnp.uint32))
  fb = lax.bitwise_or(lax.shift_right_logical(bits, _u32(9)), _u32(0x3F800000))
  f = lax.bitcast_convert_type(fb, jnp.float32) - _f32(1.0)
  u = jnp.maximum(_f32(TINY), f * _f32(1.0) + _f32(TINY))
  neg_log_u = -_log_f32(u)
  return -_log_f32(neg_log_u)


def _body(logits_hbm, out_hbm, row_v, blkmax_v, outv_v):
  cid = lax.axis_index("c")
  sid = lax.axis_index("s")
  wid = sid * 2 + cid
  iota = lax.iota(jnp.int32, 16)

  def blend_store(ref, idx, val):
    base = lax.shift_left(lax.shift_right_logical(idx, 4), 4)
    lane = idx - base
    vec = ref[pl.ds(base, 16)]
    ref[pl.ds(base, 16)] = jnp.where(iota == lane, val, vec)

  outv_v[pl.ds(0, 16)] = _i32(0)

  for j in range(ROWS_PER_TILE):
    r = wid * ROWS_PER_TILE + j
    pltpu.sync_copy(logits_hbm.at[r], row_v)

    @plsc.parallel_loop(0, (NBLK - 1) // 16, unroll=1)
    def _bm16_body(g):
      base0 = g * (16 * BLK)
      outv = _f32(NEG_INF)
      for u in range(16):
        acc = row_v[pl.ds(base0 + u * BLK, 16)]
        for t in range(1, NVEC_BLK):
          acc = jnp.maximum(acc, row_v[pl.ds(base0 + u * BLK + 16 * t, 16)])
        outv = jnp.where(iota == _i32(u), jnp.max(acc), outv)
      blkmax_v[pl.ds(g * 16, 16)] = outv / _f32(TEMPERATURE)

    acc = row_v[pl.ds((NBLK - 1) * BLK, 16)]
    for t in range(1, NVEC_BLK):
      acc = jnp.maximum(acc, row_v[pl.ds((NBLK - 1) * BLK + 16 * t, 16)])
    outv = jnp.where(iota == _i32(0), jnp.max(acc), _f32(NEG_INF))
    blkmax_v[pl.ds(NBLK - 1, 16)] = outv / _f32(TEMPERATURE)

    svs = []
    for h in range(3):
      outv = _f32(NEG_INF)
      for u in range(16):
        s = h * 16 + u
        if s < NSUP:
          outv = jnp.where(iota == _i32(u), jnp.max(blkmax_v[pl.ds(s * 16, 16)]),
                           outv)
      svs.append(outv)

    def tour_body(i, carry):
      sv0, sv1, sv2, cv0, cv1, cv2, cv3, ci0, ci1, ci2, ci3 = carry
      m = jnp.max(jnp.maximum(jnp.maximum(sv0, sv1), sv2))
      f0 = plsc.all_reduce_ffs(sv0 == m)
      f1 = plsc.all_reduce_ffs(sv1 == m)
      f2 = plsc.all_reduce_ffs(sv2 == m)
      a0 = jnp.where(f0 == _i32(16), _i32(BIG_I), f0)
      a1 = jnp.where(f1 == _i32(16), _i32(BIG_I), f1 + _i32(16))
      a2 = jnp.where(f2 == _i32(16), _i32(BIG_I), f2 + _i32(32))
      s_star = jnp.minimum(jnp.minimum(a0, a1), a2)

      bmv = plsc.load_gather(blkmax_v, [s_star * _i32(16) + iota])
      b_star = s_star * _i32(16) + plsc.all_reduce_ffs(bmv == m)

      base = b_star * _i32(BLK)
      xvs = []
      off = _i32(BIG_I)
      for t in range(NVEC_BLK):
        xv = plsc.load_gather(row_v, [base + _i32(16 * t) + iota])
        xv = xv / _f32(TEMPERATURE)
        xvs.append(xv)
        ft = plsc.all_reduce_ffs(xv == m)
        off = jnp.minimum(
            off, jnp.where(ft == _i32(16), _i32(BIG_I), ft + _i32(16 * t)))
      v_star = base + off

      plsc.store_scatter(row_v, [v_star], _f32(NEG_INF), mask=iota == _i32(0))
      accv = _f32(NEG_INF)
      for t in range(NVEC_BLK):
        accv = jnp.maximum(
            accv,
            jnp.where(iota + _i32(16 * t) == off, _f32(NEG_INF), xvs[t]))
      newmax = jnp.max(accv)
      suprest = jnp.max(
          jnp.where(iota + _i32(16) * s_star == b_star, _f32(NEG_INF), bmv))
      supnew = _f32(0.0) + jnp.maximum(suprest, newmax)
      plsc.store_scatter(blkmax_v, [b_star], _f32(0.0) + newmax,
                         mask=iota == _i32(0))
      sv0 = jnp.where(iota == s_star, supnew, sv0)
      sv1 = jnp.where(iota + _i32(16) == s_star, supnew, sv1)
      sv2 = jnp.where(iota + _i32(32) == s_star, supnew, sv2)

      mv = _f32(0.0) + m
      cv0 = jnp.where(iota == i, mv, cv0)
      cv1 = jnp.where(iota + _i32(16) == i, mv, cv1)
      cv2 = jnp.where(iota + _i32(32) == i, mv, cv2)
      cv3 = jnp.where(iota + _i32(48) == i, mv, cv3)
      ci0 = jnp.where(iota == i, v_star, ci0)
      ci1 = jnp.where(iota + _i32(16) == i, v_star, ci1)
      ci2 = jnp.where(iota + _i32(32) == i, v_star, ci2)
      ci3 = jnp.where(iota + _i32(48) == i, v_star, ci3)
      return (sv0, sv1, sv2, cv0, cv1, cv2, cv3, ci0, ci1, ci2, ci3)

    init = (svs[0], svs[1], svs[2],
            _f32(NEG_INF), _f32(NEG_INF), _f32(NEG_INF), _f32(NEG_INF),
            _i32(0), _i32(0), _i32(0), _i32(0))
    (_, _, _, cv0, cv1, cv2, cv3,
     ci0, ci1, ci2, ci3) = lax.fori_loop(0, K, tour_body, init)

    m0 = jnp.max(cv0)
    xs = [cv0, cv1, cv2, cv3]
    idxs = [ci0, ci1, ci2, ci3]
    lanes = [iota + _i32(16 * t) for t in range(4)]
    valid = [lanes[t] < _i32(K) for t in range(4)]

    es = []
    for t in range(4):
      x_safe = jnp.where(valid[t], xs[t], _f32(0.0) + m0)
      es.append(jnp.where(valid[t], jnp.exp(x_safe - m0), _f32(0.0)))
    s1 = jnp.sum(es[0]) + jnp.sum(es[1]) + jnp.sum(es[2]) + jnp.sum(es[3])

    kcount = jnp.int32(0)
    carry = jnp.float32(0.0)
    for t in range(4):
      p = es[t] / s1
      c = plsc.cumsum(p) + carry
      carry = jnp.max(c)
      cnt = plsc.all_reduce_population_count(c <= _f32(TOP_P))
      kcount = kcount + jnp.max(cnt)

    keeps = [jnp.logical_and(valid[t], lanes[t] <= kcount) for t in range(4)]
    e2s = [jnp.where(keeps[t], es[t], _f32(0.0)) for t in range(4)]
    s2 = jnp.sum(e2s[0]) + jnp.sum(e2s[1]) + jnp.sum(e2s[2]) + jnp.sum(e2s[3])

    best = _f32(NEG_INF)
    avs = []
    for t in range(4):
      q = e2s[t] / s2
      pos = q > _f32(0.0)
      q_safe = jnp.where(pos, q, _f32(1.0))
      logq = _log_f32(q_safe)
      g = _gumbel(idxs[t] + _i32(r * V))
      a = jnp.where(
          jnp.logical_and(keeps[t], pos), logq + g, _f32(NEG_INF))
      avs.append(a)
      best = jnp.maximum(best, a)
    bestv = jnp.max(best)
    win = _i32(BIG_I)
    for t in range(4):
      win = jnp.minimum(win, jnp.where(avs[t] == bestv, idxs[t], _i32(BIG_I)))
    blend_store(outv_v, jnp.int32(j), jnp.min(win))

  pltpu.sync_copy(outv_v, out_hbm.at[wid])


def kernel(logits_BCxV):
  mesh = plsc.VectorSubcoreMesh(core_axis_name="c", subcore_axis_name="s")
  run = pl.kernel(
      _body,
      out_type=jax.ShapeDtypeStruct((NTILES, 16), jnp.int32),
      mesh=mesh,
      compiler_params=pltpu.CompilerParams(needs_layout_passes=False),
      scratch_types=[
          pltpu.VMEM((V,), jnp.float32),
          pltpu.VMEM((NBLK_PAD,), jnp.float32),
          pltpu.VMEM((16,), jnp.int32),
      ],
  )
  res = run(logits_BCxV)
  return res[:, :ROWS_PER_TILE].reshape(B)
